# baseline (speedup 1.0000x reference)
import functools

import jax
import jax.numpy as jnp
from jax import lax
from jax.experimental import pallas as pl
from jax.experimental.pallas import tpu as pltpu
from jax.experimental.pallas import tpu_sc as plsc

VOCAB = 1000000
HIDDEN = 64
B = 4096
L = 200

NC = 2
NS = 16
NW = NC * NS
TOKENS = B * L
PER_W = TOKENS // NW
CHUNK = 128
NCHUNKS = PER_W // CHUNK
NBUF = 2

assert TOKENS == NW * NCHUNKS * CHUNK


def _body(ids_hbm, table_hbm, out_hbm, idx_v, r0, r1, sg0, sg1, sw0, sw1):
    wid = lax.axis_index("s") * NC + lax.axis_index("c")
    rows = (r0, r1)
    gsems = (sg0, sg1)
    wsems = (sw0, sw1)

    pltpu.sync_copy(ids_hbm.at[wid], idx_v)

    for b in range(NBUF):
        pltpu.async_copy(table_hbm.at[idx_v.at[b]], rows[b], gsems[b])

    def step(g):
        for b in range(NBUF):
            c = g + b
            pltpu.make_async_copy(
                table_hbm.at[idx_v.at[c]], rows[b], gsems[b]
            ).wait()
            pltpu.async_copy(rows[b], out_hbm.at[wid, c], wsems[b])
        for b in range(NBUF):
            c = g + b
            nxt = c + NBUF

            @pl.when(nxt < NCHUNKS)
            def _():
                pltpu.make_async_copy(
                    rows[b], out_hbm.at[wid, c], wsems[b]
                ).wait()
                pltpu.async_copy(table_hbm.at[idx_v.at[nxt]], rows[b], gsems[b])

    pl.loop(0, NCHUNKS, step=NBUF)(step)

    for b in range(NBUF):
        c = NCHUNKS - NBUF + b
        pltpu.make_async_copy(rows[b], out_hbm.at[wid, c], wsems[b]).wait()


@jax.jit
def _embed(ids, table):
    grid_kernel = pl.kernel(
        _body,
        out_type=jax.ShapeDtypeStruct((NW, NCHUNKS, CHUNK, HIDDEN), jnp.float32),
        mesh=plsc.VectorSubcoreMesh(
            core_axis_name="c", subcore_axis_name="s",
            num_cores=NC, num_subcores=NS,
        ),
        scratch_types=[
            pltpu.VMEM((NCHUNKS, CHUNK), jnp.int32),
            pltpu.VMEM((CHUNK, HIDDEN), jnp.float32),
            pltpu.VMEM((CHUNK, HIDDEN), jnp.float32),
            pltpu.SemaphoreType.DMA,
            pltpu.SemaphoreType.DMA,
            pltpu.SemaphoreType.DMA,
            pltpu.SemaphoreType.DMA,
        ],
        compiler_params=pltpu.CompilerParams(use_tc_tiling_on_sc=False),
    )
    return grid_kernel(ids, table)


def kernel(input_ids, table):
    ids = jnp.reshape(input_ids.astype(jnp.int32), (NW, NCHUNKS, CHUNK))
    out = _embed(ids, table)
    return jnp.reshape(out, (B, L, HIDDEN))


# speedup vs baseline: 1.0344x; 1.0344x over previous
import functools

import jax
import jax.numpy as jnp
from jax import lax
from jax.experimental import pallas as pl
from jax.experimental.pallas import tpu as pltpu
from jax.experimental.pallas import tpu_sc as plsc

VOCAB = 1000000
HIDDEN = 64
B = 4096
L = 200

NC = 2
NS = 16
NW = NC * NS
TOKENS = B * L
PER_W = TOKENS // NW
CHUNK = 512
NCHUNKS = PER_W // CHUNK
NBUF = 2

assert TOKENS == NW * NCHUNKS * CHUNK


def _body(ids_hbm, table_hbm, out_hbm, idx_v, r0, r1, sg0, sg1, sw0, sw1):
    wid = lax.axis_index("s") * NC + lax.axis_index("c")
    rows = (r0, r1)
    gsems = (sg0, sg1)
    wsems = (sw0, sw1)

    pltpu.sync_copy(ids_hbm.at[wid], idx_v)

    for b in range(NBUF):
        pltpu.async_copy(table_hbm.at[idx_v.at[b]], rows[b], gsems[b])

    def step(g):
        for b in range(NBUF):
            c = g + b
            pltpu.make_async_copy(
                table_hbm.at[idx_v.at[c]], rows[b], gsems[b]
            ).wait()
            pltpu.async_copy(rows[b], out_hbm.at[wid, c], wsems[b])
        for b in range(NBUF):
            c = g + b
            nxt = c + NBUF

            @pl.when(nxt < NCHUNKS)
            def _():
                pltpu.make_async_copy(
                    rows[b], out_hbm.at[wid, c], wsems[b]
                ).wait()
                pltpu.async_copy(table_hbm.at[idx_v.at[nxt]], rows[b], gsems[b])

    pl.loop(0, NCHUNKS, step=NBUF)(step)

    for b in range(NBUF):
        c = NCHUNKS - NBUF + b
        pltpu.make_async_copy(rows[b], out_hbm.at[wid, c], wsems[b]).wait()


@jax.jit
def _embed(ids, table):
    grid_kernel = pl.kernel(
        _body,
        out_type=jax.ShapeDtypeStruct((NW, NCHUNKS, CHUNK, HIDDEN), jnp.float32),
        mesh=plsc.VectorSubcoreMesh(
            core_axis_name="c", subcore_axis_name="s",
            num_cores=NC, num_subcores=NS,
        ),
        scratch_types=[
            pltpu.VMEM((NCHUNKS, CHUNK), jnp.int32),
            pltpu.VMEM((CHUNK, HIDDEN), jnp.float32),
            pltpu.VMEM((CHUNK, HIDDEN), jnp.float32),
            pltpu.SemaphoreType.DMA,
            pltpu.SemaphoreType.DMA,
            pltpu.SemaphoreType.DMA,
            pltpu.SemaphoreType.DMA,
        ],
        compiler_params=pltpu.CompilerParams(use_tc_tiling_on_sc=False),
    )
    return grid_kernel(ids, table)


def kernel(input_ids, table):
    ids = jnp.reshape(input_ids.astype(jnp.int32), (NW, NCHUNKS, CHUNK))
    out = _embed(ids, table)
    return jnp.reshape(out, (B, L, HIDDEN))


# speedup vs baseline: 1.0344x; 1.0000x over previous
import functools

import jax
import jax.numpy as jnp
from jax import lax
from jax.experimental import pallas as pl
from jax.experimental.pallas import tpu as pltpu
from jax.experimental.pallas import tpu_sc as plsc

VOCAB = 1000000
HIDDEN = 64
B = 4096
L = 200

NC = 2
NS = 16
NW = NC * NS
TOKENS = B * L
PER_W = TOKENS // NW
CHUNK = 512
NCHUNKS = PER_W // CHUNK
NBUF = 2
OUT_ROWS = TOKENS // 2

assert TOKENS == NW * NCHUNKS * CHUNK


def _body(ids_hbm, table_hbm, out_hbm, idx_v, r0, r1, sg0, sg1, sw0, sw1):
    wid = lax.axis_index("s") * NC + lax.axis_index("c")
    rows = (r0, r1)
    gsems = (sg0, sg1)
    wsems = (sw0, sw1)
    out2 = out_hbm
    wbase = wid * PER_W

    pltpu.sync_copy(ids_hbm.at[wid], idx_v)

    for b in range(NBUF):
        pltpu.async_copy(table_hbm.at[idx_v.at[b]], rows[b], gsems[b])

    def step(g):
        for b in range(NBUF):
            c = g + b
            pltpu.make_async_copy(
                table_hbm.at[idx_v.at[c]], rows[b], gsems[b]
            ).wait()
            pltpu.async_copy(
                rows[b], out2.at[pl.ds(wbase + c * CHUNK, CHUNK)], wsems[b]
            )
        for b in range(NBUF):
            c = g + b
            nxt = c + NBUF

            @pl.when(nxt < NCHUNKS)
            def _():
                pltpu.make_async_copy(
                    rows[b], out2.at[pl.ds(wbase + c * CHUNK, CHUNK)], wsems[b]
                ).wait()
                pltpu.async_copy(table_hbm.at[idx_v.at[nxt]], rows[b], gsems[b])

    pl.loop(0, NCHUNKS, step=NBUF)(step)

    for b in range(NBUF):
        c = NCHUNKS - NBUF + b
        pltpu.make_async_copy(
            rows[b], out2.at[pl.ds(wbase + c * CHUNK, CHUNK)], wsems[b]
        ).wait()


@jax.jit
def _embed(ids, table):
    grid_kernel = pl.kernel(
        _body,
        out_type=jax.ShapeDtypeStruct((TOKENS, HIDDEN), jnp.float32),
        mesh=plsc.VectorSubcoreMesh(
            core_axis_name="c", subcore_axis_name="s",
            num_cores=NC, num_subcores=NS,
        ),
        scratch_types=[
            pltpu.VMEM((NCHUNKS, CHUNK), jnp.int32),
            pltpu.VMEM((CHUNK, HIDDEN), jnp.float32),
            pltpu.VMEM((CHUNK, HIDDEN), jnp.float32),
            pltpu.SemaphoreType.DMA,
            pltpu.SemaphoreType.DMA,
            pltpu.SemaphoreType.DMA,
            pltpu.SemaphoreType.DMA,
        ],
        compiler_params=pltpu.CompilerParams(use_tc_tiling_on_sc=False),
    )
    return grid_kernel(ids, table)


def kernel(input_ids, table):
    ids = jnp.reshape(input_ids.astype(jnp.int32), (NW, NCHUNKS, CHUNK))
    out = _embed(ids, table)
    return jnp.reshape(out, (B, L, HIDDEN))
